# baseline (speedup 1.0000x reference)
import functools

import jax
import jax.numpy as jnp
from jax import lax
from jax.experimental import pallas as pl
from jax.experimental.pallas import tpu as pltpu
from jax.experimental.pallas import tpu_sc as plsc

D = 128
R = 4096
W = 50
V = 10000
NC, NS = 2, 16
NW = NC * NS
R_PER_W = R // NW
NBUF = 4
NGRP = R_PER_W // NBUF

_mesh = plsc.VectorSubcoreMesh(core_axis_name="c", subcore_axis_name="s")


@functools.partial(
    pl.kernel,
    mesh=_mesh,
    out_type=jax.ShapeDtypeStruct((R, W, D), jnp.float32),
    compiler_params=pltpu.CompilerParams(use_tc_tiling_on_sc=True),
    scratch_types=[
        pltpu.VMEM((R_PER_W, W), jnp.int32),
        pltpu.VMEM_SHARED((V, D), jnp.float32),
    ]
    + [pltpu.VMEM((W, D), jnp.float32) for _ in range(NBUF)]
    + [pltpu.SemaphoreType.DMA for _ in range(2 * NBUF)],
)
def _gather_kernel(pe_hbm, idx_hbm, out_hbm, idx_v, pe_sp, *rest):
    bufs = rest[:NBUF]
    gsems = rest[NBUF:2 * NBUF]
    ssems = rest[2 * NBUF:]

    sid = lax.axis_index("s")
    wid = sid * NC + lax.axis_index("c")
    base = wid * R_PER_W
    pltpu.sync_copy(idx_hbm.at[pl.ds(base, R_PER_W)], idx_v)

    rows = 624
    pltpu.sync_copy(
        pe_hbm.at[pl.ds(sid * rows, rows)], pe_sp.at[pl.ds(sid * rows, rows)]
    )

    @pl.when(sid == 0)
    def _():
        pltpu.sync_copy(
            pe_hbm.at[pl.ds(16 * rows, V - 16 * rows)],
            pe_sp.at[pl.ds(16 * rows, V - 16 * rows)],
        )

    plsc.subcore_barrier()

    def gather(r, b):
        return pltpu.make_async_copy(
            pe_sp.at[idx_v.at[r]], bufs[b], gsems[b]
        )

    def store(r, b):
        return pltpu.make_async_copy(bufs[b], out_hbm.at[base + r], ssems[b])

    for b in range(NBUF):
        gather(b, b).start()

    def grp(g, carry):
        r0 = g * NBUF
        for b in range(NBUF):
            gather(r0 + b, b).wait()
            store(r0 + b, b).start()
        for b in range(NBUF):
            store(r0 + b, b).wait()
            gather(r0 + NBUF + b, b).start()
        return carry

    lax.fori_loop(0, NGRP - 1, grp, 0)

    r0 = (NGRP - 1) * NBUF
    for b in range(NBUF):
        gather(r0 + b, b).wait()
        store(r0 + b, b).start()
    for b in range(NBUF):
        store(r0 + b, b).wait()


def kernel(t, pe):
    return _gather_kernel(pe, t)


# speedup vs baseline: 1.5345x; 1.5345x over previous
import functools

import jax
import jax.numpy as jnp
from jax import lax
from jax.experimental import pallas as pl
from jax.experimental.pallas import tpu as pltpu
from jax.experimental.pallas import tpu_sc as plsc

D = 128
R = 4096
W = 50
V = 10000
NC, NS = 2, 16
NW = NC * NS
R_PER_W = R // NW
NBUF = 2
NGRP = W // NBUF

_mesh = plsc.VectorSubcoreMesh(core_axis_name="c", subcore_axis_name="s")


@functools.partial(
    pl.kernel,
    mesh=_mesh,
    out_type=jax.ShapeDtypeStruct((W, R, D), jnp.float32),
    compiler_params=pltpu.CompilerParams(use_tc_tiling_on_sc=True),
    scratch_types=[
        pltpu.VMEM((W, R_PER_W), jnp.int32),
        pltpu.VMEM_SHARED((V, D), jnp.float32),
    ]
    + [pltpu.VMEM((R_PER_W, D), jnp.float32) for _ in range(NBUF)]
    + [pltpu.SemaphoreType.DMA for _ in range(2 * NBUF)],
)
def _gather_kernel(pe_hbm, idx_hbm, out_hbm, idx_v, pe_sp, *rest):
    bufs = rest[:NBUF]
    gsems = rest[NBUF:2 * NBUF]
    ssems = rest[2 * NBUF:]

    sid = lax.axis_index("s")
    wid = sid * NC + lax.axis_index("c")
    base = wid * R_PER_W
    pltpu.sync_copy(idx_hbm.at[:, pl.ds(base, R_PER_W)], idx_v)

    rows = 624
    pltpu.sync_copy(
        pe_hbm.at[pl.ds(sid * rows, rows)], pe_sp.at[pl.ds(sid * rows, rows)]
    )

    @pl.when(sid == 0)
    def _():
        pltpu.sync_copy(
            pe_hbm.at[pl.ds(16 * rows, V - 16 * rows)],
            pe_sp.at[pl.ds(16 * rows, V - 16 * rows)],
        )

    plsc.subcore_barrier()

    def gather(w, b):
        return pltpu.make_async_copy(
            pe_sp.at[idx_v.at[w]], bufs[b], gsems[b]
        )

    def store(w, b):
        return pltpu.make_async_copy(
            bufs[b], out_hbm.at[w, pl.ds(base, R_PER_W)], ssems[b]
        )

    for b in range(NBUF):
        gather(b, b).start()

    def grp(g, carry):
        w0 = g * NBUF
        for b in range(NBUF):
            gather(w0 + b, b).wait()
            store(w0 + b, b).start()
        for b in range(NBUF):
            store(w0 + b, b).wait()
            gather(w0 + NBUF + b, b).start()
        return carry

    lax.fori_loop(0, NGRP - 1, grp, 0)

    w0 = (NGRP - 1) * NBUF
    for b in range(NBUF):
        gather(w0 + b, b).wait()
        store(w0 + b, b).start()
    for b in range(NBUF):
        store(w0 + b, b).wait()


def kernel(t, pe):
    outT = _gather_kernel(pe, t.T)
    return outT.transpose(1, 0, 2)
